# Initial kernel scaffold; baseline (speedup 1.0000x reference)
import jax
import jax.numpy as jnp
from jax.experimental import pallas as pl


def kernel(x, edge_index, batch, emb, W1, b1, ln1_g, ln1_b, W2, b2, ln2_g, ln2_b, Wg1, bg1, Wg2, bg2, Wc1, bc1, Wc2, bc2):
    raise NotImplementedError("write your pallas kernel here")


# speedup vs baseline: 9.5466x; 9.5466x over previous
import jax
import jax.numpy as jnp
from jax import lax
from jax.experimental import pallas as pl
from jax.experimental.pallas import tpu as pltpu
from jax.experimental.pallas import tpu_sc as plsc

N = 10000
E = 320000
G = 64
DIM = 256
HDIM = 128
BLK = 200
NB = N // BLK

_mesh_cache = {}


def _mesh():
    if "m" not in _mesh_cache:
        _mesh_cache["m"] = plsc.VectorSubcoreMesh(
            core_axis_name="c", subcore_axis_name="s",
            num_cores=2, num_subcores=16)
    return _mesh_cache["m"]


def _deg_body(ones_hbm, edge_hbm, out_hbm, idx_d, rows, hist):
    c = lax.axis_index("c")
    s = lax.axis_index("s")

    def zero_rows(i, _):
        rows[i // 8, pl.ds((i % 8) * 16, 16)] = jnp.zeros((16,), jnp.float32)
        return 0
    lax.fori_loop(0, 1024, zero_rows, 0)

    def zero_acc(j, _):
        pltpu.sync_copy(rows.at[pl.ds(0, 104)],
                        hist.at[pl.ds(s * 624 + j * 104, 104)])
        return 0
    lax.fori_loop(0, 6, zero_acc, 0)

    @pl.when(s == 0)
    def _():
        pltpu.sync_copy(rows.at[pl.ds(0, 16)], hist.at[pl.ds(9984, 16)])

    plsc.subcore_barrier()

    pltpu.sync_copy(ones_hbm.at[pl.ds(0, 128)], rows)

    k = c * 16 + s

    def chunk(j, _):
        off = (j * 32 + k) * 128
        pltpu.sync_copy(edge_hbm.at[pl.ds(E + off, 128)], idx_d)
        pltpu.sync_copy(rows, hist.at[idx_d], add=True)
        return 0
    lax.fori_loop(0, 78, chunk, 0)

    @pl.when(k < 4)
    def _():
        off = (2496 + k) * 128
        pltpu.sync_copy(edge_hbm.at[pl.ds(E + off, 128)], idx_d)
        pltpu.sync_copy(rows, hist.at[idx_d], add=True)

    plsc.subcore_barrier()

    def copy_out(j, _):
        pltpu.sync_copy(hist.at[pl.ds(s * 624 + j * 104, 104)],
                        rows.at[pl.ds(0, 104)])
        pltpu.sync_copy(rows.at[pl.ds(0, 104)],
                        out_hbm.at[pl.ds(c * N + s * 624 + j * 104, 104)])
        return 0
    lax.fori_loop(0, 6, copy_out, 0)

    @pl.when(s == 0)
    def _():
        pltpu.sync_copy(hist.at[pl.ds(9984, 16)], rows.at[pl.ds(0, 16)])
        pltpu.sync_copy(rows.at[pl.ds(0, 16)],
                        out_hbm.at[pl.ds(c * N + 9984, 16)])


def _deg(ones, edges):
    return pl.kernel(
        _deg_body,
        out_type=jax.ShapeDtypeStruct((2 * N, HDIM), jnp.float32),
        mesh=_mesh(),
        scratch_types=[
            pltpu.VMEM((128,), jnp.int32),
            pltpu.VMEM((128, HDIM), jnp.float32),
            pltpu.VMEM_SHARED((N, HDIM), jnp.float32),
        ],
    )(ones, edges)


def _spmm_body(tlo_hbm, thi_hbm, edge_hbm, out_hbm,
               idx_src, idx_dst, rows, idx_src_t, idx_dst_t, rows_t, acc):
    c = lax.axis_index("c")
    s = lax.axis_index("s")

    def zero_rows(i, _):
        rows[i // 8, pl.ds((i % 8) * 16, 16)] = jnp.zeros((16,), jnp.float32)
        return 0
    lax.fori_loop(0, 1024, zero_rows, 0)

    def zero_acc(j, _):
        pltpu.sync_copy(rows.at[pl.ds(0, 104)],
                        acc.at[pl.ds(s * 624 + j * 104, 104)])
        return 0
    lax.fori_loop(0, 6, zero_acc, 0)

    @pl.when(s == 0)
    def _():
        pltpu.sync_copy(rows.at[pl.ds(0, 16)], acc.at[pl.ds(9984, 16)])

    plsc.subcore_barrier()

    base = s * 20000

    def chunk(i, _):
        off = base + i * 128
        pltpu.sync_copy(edge_hbm.at[pl.ds(off, 128)], idx_src)
        pltpu.sync_copy(edge_hbm.at[pl.ds(E + off, 128)], idx_dst)

        @pl.when(c == 0)
        def _():
            pltpu.sync_copy(tlo_hbm.at[idx_src], rows)

        @pl.when(c == 1)
        def _():
            pltpu.sync_copy(thi_hbm.at[idx_src], rows)

        pltpu.sync_copy(rows, acc.at[idx_dst], add=True)
        return 0
    lax.fori_loop(0, 156, chunk, 0)

    off = base + 19968
    pltpu.sync_copy(edge_hbm.at[pl.ds(off, 32)], idx_src_t)
    pltpu.sync_copy(edge_hbm.at[pl.ds(E + off, 32)], idx_dst_t)

    @pl.when(c == 0)
    def _():
        pltpu.sync_copy(tlo_hbm.at[idx_src_t], rows_t)

    @pl.when(c == 1)
    def _():
        pltpu.sync_copy(thi_hbm.at[idx_src_t], rows_t)

    pltpu.sync_copy(rows_t, acc.at[idx_dst_t], add=True)
    plsc.subcore_barrier()

    def copy_out(j, _):
        pltpu.sync_copy(acc.at[pl.ds(s * 624 + j * 104, 104)],
                        rows.at[pl.ds(0, 104)])
        pltpu.sync_copy(rows.at[pl.ds(0, 104)],
                        out_hbm.at[pl.ds(c * N + s * 624 + j * 104, 104)])
        return 0
    lax.fori_loop(0, 6, copy_out, 0)

    @pl.when(s == 0)
    def _():
        pltpu.sync_copy(acc.at[pl.ds(9984, 16)], rows_t.at[pl.ds(0, 16)])
        pltpu.sync_copy(rows_t.at[pl.ds(0, 16)],
                        out_hbm.at[pl.ds(c * N + 9984, 16)])


def _spmm(tlo, thi, edges):
    return pl.kernel(
        _spmm_body,
        out_type=jax.ShapeDtypeStruct((2 * N, HDIM), jnp.float32),
        mesh=_mesh(),
        scratch_types=[
            pltpu.VMEM((128,), jnp.int32),
            pltpu.VMEM((128,), jnp.int32),
            pltpu.VMEM((128, HDIM), jnp.float32),
            pltpu.VMEM((32,), jnp.int32),
            pltpu.VMEM((32,), jnp.int32),
            pltpu.VMEM((32, HDIM), jnp.float32),
            pltpu.VMEM_SHARED((N, HDIM), jnp.float32),
        ],
    )(tlo, thi, edges)


def _ka_body(x_ref, d0_ref, d1_ref, emb_ref, w1a_ref, w1b_ref,
             lo_ref, hi_ref, dis_ref):
    nt = x_ref[:, 0:1].astype(jnp.int32)
    feats = x_ref[:, 1:]
    oh = (nt == lax.broadcasted_iota(jnp.int32, (BLK, 400), 1)
          ).astype(jnp.float32)
    embp = jnp.dot(oh, emb_ref[...], preferred_element_type=jnp.float32)
    hw = (jnp.dot(feats, w1a_ref[...], preferred_element_type=jnp.float32)
          + jnp.dot(embp, w1b_ref[...], preferred_element_type=jnp.float32))
    deg = 1.0 + d0_ref[:, 0:1] + d1_ref[:, 0:1]
    dis = lax.rsqrt(deg)
    hwp = dis * hw
    lo_ref[...] = hwp[:, :HDIM]
    hi_ref[...] = hwp[:, HDIM:]
    dis_ref[...] = dis


def _ka(x, degp, emb, w1a, w1b):
    return pl.pallas_call(
        _ka_body,
        grid=(NB,),
        in_specs=[
            pl.BlockSpec((BLK, 113), lambda b: (b, 0)),
            pl.BlockSpec((BLK, HDIM), lambda b: (b, 0)),
            pl.BlockSpec((BLK, HDIM), lambda b: (b + NB, 0)),
            pl.BlockSpec((400, 16), lambda b: (0, 0)),
            pl.BlockSpec((112, DIM), lambda b: (0, 0)),
            pl.BlockSpec((16, DIM), lambda b: (0, 0)),
        ],
        out_specs=[
            pl.BlockSpec((BLK, HDIM), lambda b: (b, 0)),
            pl.BlockSpec((BLK, HDIM), lambda b: (b, 0)),
            pl.BlockSpec((BLK, 1), lambda b: (b, 0)),
        ],
        out_shape=[
            jax.ShapeDtypeStruct((N, HDIM), jnp.float32),
            jax.ShapeDtypeStruct((N, HDIM), jnp.float32),
            jax.ShapeDtypeStruct((N, 1), jnp.float32),
        ],
    )(x, degp, degp, emb, w1a, w1b)


def _kb_body(alo_ref, ahi_ref, plo_ref, phi_ref, dis_ref, b1_ref, g_ref,
             be_ref, w2_ref, lo_ref, hi_ref):
    dis = dis_ref[...]
    acc = jnp.concatenate([alo_ref[...], ahi_ref[...]], axis=1)
    hwp = jnp.concatenate([plo_ref[...], phi_ref[...]], axis=1)
    t = dis * (acc + hwp) + b1_ref[...]
    mu = jnp.mean(t, axis=1, keepdims=True)
    var = jnp.mean((t - mu) ** 2, axis=1, keepdims=True)
    y = (t - mu) * lax.rsqrt(var + 1e-5) * g_ref[...] + be_ref[...]
    h = jnp.maximum(y, 0.0)
    hwp2 = dis * jnp.dot(h, w2_ref[...], preferred_element_type=jnp.float32)
    lo_ref[...] = hwp2[:, :HDIM]
    hi_ref[...] = hwp2[:, HDIM:]


def _kb(accp, plo, phi, dis, b1r, g1r, be1r, W2):
    return pl.pallas_call(
        _kb_body,
        grid=(NB,),
        in_specs=[
            pl.BlockSpec((BLK, HDIM), lambda b: (b, 0)),
            pl.BlockSpec((BLK, HDIM), lambda b: (b + NB, 0)),
            pl.BlockSpec((BLK, HDIM), lambda b: (b, 0)),
            pl.BlockSpec((BLK, HDIM), lambda b: (b, 0)),
            pl.BlockSpec((BLK, 1), lambda b: (b, 0)),
            pl.BlockSpec((1, DIM), lambda b: (0, 0)),
            pl.BlockSpec((1, DIM), lambda b: (0, 0)),
            pl.BlockSpec((1, DIM), lambda b: (0, 0)),
            pl.BlockSpec((DIM, DIM), lambda b: (0, 0)),
        ],
        out_specs=[
            pl.BlockSpec((BLK, HDIM), lambda b: (b, 0)),
            pl.BlockSpec((BLK, HDIM), lambda b: (b, 0)),
        ],
        out_shape=[
            jax.ShapeDtypeStruct((N, HDIM), jnp.float32),
            jax.ShapeDtypeStruct((N, HDIM), jnp.float32),
        ],
    )(accp, accp, plo, phi, dis, b1r, g1r, be1r, W2)


def _kc_body(alo_ref, ahi_ref, plo_ref, phi_ref, dis_ref, batch_ref, b2_ref,
             g_ref, be_ref, wg1_ref, bg1_ref, wg2_ref, bg2_ref,
             h2_ref, gate_ref, m_ref):
    b = pl.program_id(0)
    dis = dis_ref[...]
    acc = jnp.concatenate([alo_ref[...], ahi_ref[...]], axis=1)
    hwp = jnp.concatenate([plo_ref[...], phi_ref[...]], axis=1)
    t = dis * (acc + hwp) + b2_ref[...]
    mu = jnp.mean(t, axis=1, keepdims=True)
    var = jnp.mean((t - mu) ** 2, axis=1, keepdims=True)
    y = (t - mu) * lax.rsqrt(var + 1e-5) * g_ref[...] + be_ref[...]
    h2 = jnp.maximum(y, 0.0)
    h2_ref[...] = h2
    g1 = jnp.maximum(
        jnp.dot(h2, wg1_ref[...], preferred_element_type=jnp.float32)
        + bg1_ref[...], 0.0)
    gate = (jnp.dot(g1, wg2_ref[...], preferred_element_type=jnp.float32)
            + bg2_ref[...])
    gate_ref[...] = gate
    oh = batch_ref[...] == lax.broadcasted_iota(jnp.int32, (BLK, G), 1)
    masked = jnp.where(oh, gate, -jnp.inf)
    bm = jnp.max(masked, axis=0, keepdims=True)

    @pl.when(b == 0)
    def _():
        m_ref[...] = jnp.full((1, G), -jnp.inf, jnp.float32)

    m_ref[...] = jnp.maximum(m_ref[...], bm)


def _kc(accp, plo, phi, dis, batch2, b2r, g2r, be2r, Wg1, bg1r, Wg2, bg2r):
    return pl.pallas_call(
        _kc_body,
        grid=(NB,),
        in_specs=[
            pl.BlockSpec((BLK, HDIM), lambda b: (b, 0)),
            pl.BlockSpec((BLK, HDIM), lambda b: (b + NB, 0)),
            pl.BlockSpec((BLK, HDIM), lambda b: (b, 0)),
            pl.BlockSpec((BLK, HDIM), lambda b: (b, 0)),
            pl.BlockSpec((BLK, 1), lambda b: (b, 0)),
            pl.BlockSpec((BLK, 1), lambda b: (b, 0)),
            pl.BlockSpec((1, DIM), lambda b: (0, 0)),
            pl.BlockSpec((1, DIM), lambda b: (0, 0)),
            pl.BlockSpec((1, DIM), lambda b: (0, 0)),
            pl.BlockSpec((DIM, 128), lambda b: (0, 0)),
            pl.BlockSpec((1, 128), lambda b: (0, 0)),
            pl.BlockSpec((128, 1), lambda b: (0, 0)),
            pl.BlockSpec((1, 1), lambda b: (0, 0)),
        ],
        out_specs=[
            pl.BlockSpec((BLK, DIM), lambda b: (b, 0)),
            pl.BlockSpec((BLK, 1), lambda b: (b, 0)),
            pl.BlockSpec((1, G), lambda b: (0, 0)),
        ],
        out_shape=[
            jax.ShapeDtypeStruct((N, DIM), jnp.float32),
            jax.ShapeDtypeStruct((N, 1), jnp.float32),
            jax.ShapeDtypeStruct((1, G), jnp.float32),
        ],
    )(accp, accp, plo, phi, dis, batch2, b2r, g2r, be2r,
      Wg1, bg1r, Wg2, bg2r)


def _kd_body(h2_ref, gate_ref, batch_ref, m_ref, wc1_ref, bc1_ref,
             wc2_ref, bc2_ref, out_ref, s_s, p_s):
    b = pl.program_id(0)

    @pl.when(b == 0)
    def _():
        s_s[...] = jnp.zeros((1, G), jnp.float32)
        p_s[...] = jnp.zeros((G, DIM), jnp.float32)

    oh = batch_ref[...] == lax.broadcasted_iota(jnp.int32, (BLK, G), 1)
    e = jnp.where(oh, jnp.exp(gate_ref[...] - m_ref[...]), 0.0)
    s_s[...] += jnp.sum(e, axis=0, keepdims=True)
    p_s[...] += lax.dot_general(e, h2_ref[...], (((0,), (0,)), ((), ())),
                                preferred_element_type=jnp.float32)

    @pl.when(b == NB - 1)
    def _():
        sv = s_s[...]
        rs = 1.0 / jnp.where(sv > 0, sv, 1.0)
        ii = lax.broadcasted_iota(jnp.int32, (G, G), 0)
        jj = lax.broadcasted_iota(jnp.int32, (G, G), 1)
        dm = jnp.where(ii == jj, rs, 0.0)
        pooled = jnp.dot(dm, p_s[...], preferred_element_type=jnp.float32)
        q = jnp.maximum(
            jnp.dot(pooled, wc1_ref[...], preferred_element_type=jnp.float32)
            + bc1_ref[...], 0.0)
        out_ref[...] = (jnp.dot(q, wc2_ref[...],
                                preferred_element_type=jnp.float32)
                        + bc2_ref[...])


def _kd(h2, gate, batch2, m, Wc1, bc1r, Wc2, bc2r):
    return pl.pallas_call(
        _kd_body,
        grid=(NB,),
        in_specs=[
            pl.BlockSpec((BLK, DIM), lambda b: (b, 0)),
            pl.BlockSpec((BLK, 1), lambda b: (b, 0)),
            pl.BlockSpec((BLK, 1), lambda b: (b, 0)),
            pl.BlockSpec((1, G), lambda b: (0, 0)),
            pl.BlockSpec((DIM, 128), lambda b: (0, 0)),
            pl.BlockSpec((1, 128), lambda b: (0, 0)),
            pl.BlockSpec((128, 2), lambda b: (0, 0)),
            pl.BlockSpec((1, 2), lambda b: (0, 0)),
        ],
        out_specs=pl.BlockSpec((G, 2), lambda b: (0, 0)),
        out_shape=jax.ShapeDtypeStruct((G, 2), jnp.float32),
        scratch_shapes=[
            pltpu.VMEM((1, G), jnp.float32),
            pltpu.VMEM((G, DIM), jnp.float32),
        ],
    )(h2, gate, batch2, m, Wc1, bc1r, Wc2, bc2r)


def kernel(x, edge_index, batch, emb, W1, b1, ln1_g, ln1_b, W2, b2,
           ln2_g, ln2_b, Wg1, bg1, Wg2, bg2, Wc1, bc1, Wc2, bc2):
    edges = edge_index.reshape(2 * E)
    batch2 = batch.astype(jnp.int32).reshape(N, 1)
    w1a = W1[:112]
    w1b = W1[112:]
    b1r = b1.reshape(1, DIM)
    g1r = ln1_g.reshape(1, DIM)
    be1r = ln1_b.reshape(1, DIM)
    b2r = b2.reshape(1, DIM)
    g2r = ln2_g.reshape(1, DIM)
    be2r = ln2_b.reshape(1, DIM)
    bg1r = bg1.reshape(1, 128)
    bg2r = bg2.reshape(1, 1)
    bc1r = bc1.reshape(1, 128)
    bc2r = bc2.reshape(1, 2)

    degp = _deg(jnp.ones((128, HDIM), jnp.float32), edges)
    plo1, phi1, dis = _ka(x, degp, emb, w1a, w1b)
    accp1 = _spmm(plo1, phi1, edges)
    plo2, phi2 = _kb(accp1, plo1, phi1, dis, b1r, g1r, be1r, W2)
    accp2 = _spmm(plo2, phi2, edges)
    h2, gate, m = _kc(accp2, plo2, phi2, dis, batch2, b2r, g2r, be2r,
                      Wg1, bg1r, Wg2, bg2r)
    return _kd(h2, gate, batch2, m, Wc1, bc1r, Wc2, bc2r)


# speedup vs baseline: 12.0602x; 1.2633x over previous
import jax
import jax.numpy as jnp
from jax import lax
from jax.experimental import pallas as pl
from jax.experimental.pallas import tpu as pltpu
from jax.experimental.pallas import tpu_sc as plsc

N = 10000
E = 320000
G = 64
DIM = 256
HDIM = 128
BLK = 200
NB = N // BLK

_mesh_cache = {}


def _mesh():
    if "m" not in _mesh_cache:
        _mesh_cache["m"] = plsc.VectorSubcoreMesh(
            core_axis_name="c", subcore_axis_name="s",
            num_cores=2, num_subcores=16)
    return _mesh_cache["m"]


def _deg_body(ones_hbm, edge_hbm, out_hbm, idx_d, rows, hist):
    c = lax.axis_index("c")
    s = lax.axis_index("s")

    def zero_rows(i, _):
        rows[i // 8, pl.ds((i % 8) * 16, 16)] = jnp.zeros((16,), jnp.float32)
        return 0
    lax.fori_loop(0, 1024, zero_rows, 0)

    def zero_acc(j, _):
        pltpu.sync_copy(rows.at[pl.ds(0, 104)],
                        hist.at[pl.ds(s * 624 + j * 104, 104)])
        return 0
    lax.fori_loop(0, 6, zero_acc, 0)

    @pl.when(s == 0)
    def _():
        pltpu.sync_copy(rows.at[pl.ds(0, 16)], hist.at[pl.ds(9984, 16)])

    plsc.subcore_barrier()

    pltpu.sync_copy(ones_hbm.at[pl.ds(0, 128)], rows)

    k = c * 16 + s

    def chunk(j, _):
        off = (j * 32 + k) * 128
        pltpu.sync_copy(edge_hbm.at[pl.ds(E + off, 128)], idx_d)
        pltpu.sync_copy(rows, hist.at[idx_d], add=True)
        return 0
    lax.fori_loop(0, 78, chunk, 0)

    @pl.when(k < 4)
    def _():
        off = (2496 + k) * 128
        pltpu.sync_copy(edge_hbm.at[pl.ds(E + off, 128)], idx_d)
        pltpu.sync_copy(rows, hist.at[idx_d], add=True)

    plsc.subcore_barrier()

    def copy_out(j, _):
        pltpu.sync_copy(hist.at[pl.ds(s * 624 + j * 104, 104)],
                        rows.at[pl.ds(0, 104)])
        pltpu.sync_copy(rows.at[pl.ds(0, 104)],
                        out_hbm.at[pl.ds(c * N + s * 624 + j * 104, 104)])
        return 0
    lax.fori_loop(0, 6, copy_out, 0)

    @pl.when(s == 0)
    def _():
        pltpu.sync_copy(hist.at[pl.ds(9984, 16)], rows.at[pl.ds(0, 16)])
        pltpu.sync_copy(rows.at[pl.ds(0, 16)],
                        out_hbm.at[pl.ds(c * N + 9984, 16)])


def _deg(ones, edges):
    return pl.kernel(
        _deg_body,
        out_type=jax.ShapeDtypeStruct((2 * N, HDIM), jnp.float32),
        mesh=_mesh(),
        scratch_types=[
            pltpu.VMEM((128,), jnp.int32),
            pltpu.VMEM((128, HDIM), jnp.float32),
            pltpu.VMEM_SHARED((N, HDIM), jnp.float32),
        ],
    )(ones, edges)


def _spmm_body(tbl_hbm, edge_hbm, out_hbm,
               idx_sa, idx_da, idx_sb, idx_db, rows_a, rows_b,
               idx_src_t, idx_dst_t, rows_t, acc, gsa, gsb, ssa, ssb):
    c = lax.axis_index("c")
    s = lax.axis_index("s")

    def zero_rows(i, _):
        rows_a[i // 8, pl.ds((i % 8) * 16, 16)] = jnp.zeros((16,), jnp.float32)
        return 0
    lax.fori_loop(0, 1024, zero_rows, 0)

    def zero_acc(j, _):
        pltpu.sync_copy(rows_a.at[pl.ds(0, 104)],
                        acc.at[pl.ds(s * 624 + j * 104, 104)])
        return 0
    lax.fori_loop(0, 6, zero_acc, 0)

    @pl.when(s == 0)
    def _():
        pltpu.sync_copy(rows_a.at[pl.ds(0, 16)], acc.at[pl.ds(9984, 16)])

    plsc.subcore_barrier()

    base = s * 20000
    cn = c * N

    def bias(idx_ref):
        def one(k, _):
            idx_ref[pl.ds(k * 16, 16)] = idx_ref[pl.ds(k * 16, 16)] + cn
            return 0
        lax.fori_loop(0, 8, one, 0)

    def pair(j, _):
        offa = base + (2 * j) * 128
        offb = offa + 128
        pltpu.sync_copy(edge_hbm.at[pl.ds(offa, 128)], idx_sa)
        pltpu.sync_copy(edge_hbm.at[pl.ds(E + offa, 128)], idx_da)
        bias(idx_sa)
        fa = pltpu.async_copy(tbl_hbm.at[idx_sa], rows_a, gsa)
        pltpu.sync_copy(edge_hbm.at[pl.ds(offb, 128)], idx_sb)
        pltpu.sync_copy(edge_hbm.at[pl.ds(E + offb, 128)], idx_db)
        bias(idx_sb)
        fb = pltpu.async_copy(tbl_hbm.at[idx_sb], rows_b, gsb)
        fa.wait()
        sa = pltpu.async_copy(rows_a, acc.at[idx_da], ssa, add=True)
        fb.wait()
        sb = pltpu.async_copy(rows_b, acc.at[idx_db], ssb, add=True)
        sa.wait()
        sb.wait()
        return 0
    lax.fori_loop(0, 78, pair, 0)

    off = base + 19968
    pltpu.sync_copy(edge_hbm.at[pl.ds(off, 32)], idx_src_t)
    pltpu.sync_copy(edge_hbm.at[pl.ds(E + off, 32)], idx_dst_t)

    def bias_t(k, _):
        idx_src_t[pl.ds(k * 16, 16)] = idx_src_t[pl.ds(k * 16, 16)] + cn
        return 0
    lax.fori_loop(0, 2, bias_t, 0)
    pltpu.sync_copy(tbl_hbm.at[idx_src_t], rows_t)
    pltpu.sync_copy(rows_t, acc.at[idx_dst_t], add=True)
    plsc.subcore_barrier()

    def copy_out(j, _):
        pltpu.sync_copy(acc.at[pl.ds(s * 624 + j * 104, 104)],
                        rows_a.at[pl.ds(0, 104)])
        pltpu.sync_copy(rows_a.at[pl.ds(0, 104)],
                        out_hbm.at[pl.ds(c * N + s * 624 + j * 104, 104)])
        return 0
    lax.fori_loop(0, 6, copy_out, 0)

    @pl.when(s == 0)
    def _():
        pltpu.sync_copy(acc.at[pl.ds(9984, 16)], rows_t.at[pl.ds(0, 16)])
        pltpu.sync_copy(rows_t.at[pl.ds(0, 16)],
                        out_hbm.at[pl.ds(c * N + 9984, 16)])


def _spmm(tbl, edges):
    return pl.kernel(
        _spmm_body,
        out_type=jax.ShapeDtypeStruct((2 * N, HDIM), jnp.float32),
        mesh=_mesh(),
        scratch_types=[
            pltpu.VMEM((128,), jnp.int32),
            pltpu.VMEM((128,), jnp.int32),
            pltpu.VMEM((128,), jnp.int32),
            pltpu.VMEM((128,), jnp.int32),
            pltpu.VMEM((128, HDIM), jnp.float32),
            pltpu.VMEM((128, HDIM), jnp.float32),
            pltpu.VMEM((32,), jnp.int32),
            pltpu.VMEM((32,), jnp.int32),
            pltpu.VMEM((32, HDIM), jnp.float32),
            pltpu.VMEM_SHARED((N, HDIM), jnp.float32),
            pltpu.SemaphoreType.DMA,
            pltpu.SemaphoreType.DMA,
            pltpu.SemaphoreType.DMA,
            pltpu.SemaphoreType.DMA,
        ],
    )(tbl, edges)


def _ka_body(x_ref, d0_ref, d1_ref, emb_ref, w1a_ref, w1b_ref,
             lo_ref, hi_ref, dis_ref):
    nt = x_ref[:, 0:1].astype(jnp.int32)
    feats = x_ref[:, 1:]
    oh = (nt == lax.broadcasted_iota(jnp.int32, (BLK, 400), 1)
          ).astype(jnp.float32)
    embp = jnp.dot(oh, emb_ref[...], preferred_element_type=jnp.float32)
    hw = (jnp.dot(feats, w1a_ref[...], preferred_element_type=jnp.float32)
          + jnp.dot(embp, w1b_ref[...], preferred_element_type=jnp.float32))
    deg = 1.0 + d0_ref[:, 0:1] + d1_ref[:, 0:1]
    dis = lax.rsqrt(deg)
    hwp = dis * hw
    lo_ref[...] = hwp[:, :HDIM]
    hi_ref[...] = hwp[:, HDIM:]
    dis_ref[...] = dis


def _ka(x, degp, emb, w1a, w1b):
    return pl.pallas_call(
        _ka_body,
        grid=(NB,),
        in_specs=[
            pl.BlockSpec((BLK, 113), lambda b: (b, 0)),
            pl.BlockSpec((BLK, HDIM), lambda b: (b, 0)),
            pl.BlockSpec((BLK, HDIM), lambda b: (b + NB, 0)),
            pl.BlockSpec((400, 16), lambda b: (0, 0)),
            pl.BlockSpec((112, DIM), lambda b: (0, 0)),
            pl.BlockSpec((16, DIM), lambda b: (0, 0)),
        ],
        out_specs=[
            pl.BlockSpec((BLK, HDIM), lambda b: (b, 0)),
            pl.BlockSpec((BLK, HDIM), lambda b: (b, 0)),
            pl.BlockSpec((BLK, 1), lambda b: (b, 0)),
        ],
        out_shape=[
            jax.ShapeDtypeStruct((N, HDIM), jnp.float32),
            jax.ShapeDtypeStruct((N, HDIM), jnp.float32),
            jax.ShapeDtypeStruct((N, 1), jnp.float32),
        ],
    )(x, degp, degp, emb, w1a, w1b)


def _kb_body(alo_ref, ahi_ref, plo_ref, phi_ref, dis_ref, b1_ref, g_ref,
             be_ref, w2_ref, lo_ref, hi_ref):
    dis = dis_ref[...]
    acc = jnp.concatenate([alo_ref[...], ahi_ref[...]], axis=1)
    hwp = jnp.concatenate([plo_ref[...], phi_ref[...]], axis=1)
    t = dis * (acc + hwp) + b1_ref[...]
    mu = jnp.mean(t, axis=1, keepdims=True)
    var = jnp.mean((t - mu) ** 2, axis=1, keepdims=True)
    y = (t - mu) * lax.rsqrt(var + 1e-5) * g_ref[...] + be_ref[...]
    h = jnp.maximum(y, 0.0)
    hwp2 = dis * jnp.dot(h, w2_ref[...], preferred_element_type=jnp.float32)
    lo_ref[...] = hwp2[:, :HDIM]
    hi_ref[...] = hwp2[:, HDIM:]


def _kb(accp, plo, phi, dis, b1r, g1r, be1r, W2):
    return pl.pallas_call(
        _kb_body,
        grid=(NB,),
        in_specs=[
            pl.BlockSpec((BLK, HDIM), lambda b: (b, 0)),
            pl.BlockSpec((BLK, HDIM), lambda b: (b + NB, 0)),
            pl.BlockSpec((BLK, HDIM), lambda b: (b, 0)),
            pl.BlockSpec((BLK, HDIM), lambda b: (b, 0)),
            pl.BlockSpec((BLK, 1), lambda b: (b, 0)),
            pl.BlockSpec((1, DIM), lambda b: (0, 0)),
            pl.BlockSpec((1, DIM), lambda b: (0, 0)),
            pl.BlockSpec((1, DIM), lambda b: (0, 0)),
            pl.BlockSpec((DIM, DIM), lambda b: (0, 0)),
        ],
        out_specs=[
            pl.BlockSpec((BLK, HDIM), lambda b: (b, 0)),
            pl.BlockSpec((BLK, HDIM), lambda b: (b, 0)),
        ],
        out_shape=[
            jax.ShapeDtypeStruct((N, HDIM), jnp.float32),
            jax.ShapeDtypeStruct((N, HDIM), jnp.float32),
        ],
    )(accp, accp, plo, phi, dis, b1r, g1r, be1r, W2)


def _kc_body(alo_ref, ahi_ref, plo_ref, phi_ref, dis_ref, batch_ref, b2_ref,
             g_ref, be_ref, wg1_ref, bg1_ref, wg2_ref, bg2_ref,
             h2_ref, gate_ref, m_ref):
    b = pl.program_id(0)
    dis = dis_ref[...]
    acc = jnp.concatenate([alo_ref[...], ahi_ref[...]], axis=1)
    hwp = jnp.concatenate([plo_ref[...], phi_ref[...]], axis=1)
    t = dis * (acc + hwp) + b2_ref[...]
    mu = jnp.mean(t, axis=1, keepdims=True)
    var = jnp.mean((t - mu) ** 2, axis=1, keepdims=True)
    y = (t - mu) * lax.rsqrt(var + 1e-5) * g_ref[...] + be_ref[...]
    h2 = jnp.maximum(y, 0.0)
    h2_ref[...] = h2
    g1 = jnp.maximum(
        jnp.dot(h2, wg1_ref[...], preferred_element_type=jnp.float32)
        + bg1_ref[...], 0.0)
    gate = (jnp.dot(g1, wg2_ref[...], preferred_element_type=jnp.float32)
            + bg2_ref[...])
    gate_ref[...] = gate
    oh = batch_ref[...] == lax.broadcasted_iota(jnp.int32, (BLK, G), 1)
    masked = jnp.where(oh, gate, -jnp.inf)
    bm = jnp.max(masked, axis=0, keepdims=True)

    @pl.when(b == 0)
    def _():
        m_ref[...] = jnp.full((1, G), -jnp.inf, jnp.float32)

    m_ref[...] = jnp.maximum(m_ref[...], bm)


def _kc(accp, plo, phi, dis, batch2, b2r, g2r, be2r, Wg1, bg1r, Wg2, bg2r):
    return pl.pallas_call(
        _kc_body,
        grid=(NB,),
        in_specs=[
            pl.BlockSpec((BLK, HDIM), lambda b: (b, 0)),
            pl.BlockSpec((BLK, HDIM), lambda b: (b + NB, 0)),
            pl.BlockSpec((BLK, HDIM), lambda b: (b, 0)),
            pl.BlockSpec((BLK, HDIM), lambda b: (b, 0)),
            pl.BlockSpec((BLK, 1), lambda b: (b, 0)),
            pl.BlockSpec((BLK, 1), lambda b: (b, 0)),
            pl.BlockSpec((1, DIM), lambda b: (0, 0)),
            pl.BlockSpec((1, DIM), lambda b: (0, 0)),
            pl.BlockSpec((1, DIM), lambda b: (0, 0)),
            pl.BlockSpec((DIM, 128), lambda b: (0, 0)),
            pl.BlockSpec((1, 128), lambda b: (0, 0)),
            pl.BlockSpec((128, 1), lambda b: (0, 0)),
            pl.BlockSpec((1, 1), lambda b: (0, 0)),
        ],
        out_specs=[
            pl.BlockSpec((BLK, DIM), lambda b: (b, 0)),
            pl.BlockSpec((BLK, 1), lambda b: (b, 0)),
            pl.BlockSpec((1, G), lambda b: (0, 0)),
        ],
        out_shape=[
            jax.ShapeDtypeStruct((N, DIM), jnp.float32),
            jax.ShapeDtypeStruct((N, 1), jnp.float32),
            jax.ShapeDtypeStruct((1, G), jnp.float32),
        ],
    )(accp, accp, plo, phi, dis, batch2, b2r, g2r, be2r,
      Wg1, bg1r, Wg2, bg2r)


def _kd_body(h2_ref, gate_ref, batch_ref, m_ref, wc1_ref, bc1_ref,
             wc2_ref, bc2_ref, out_ref, s_s, p_s):
    b = pl.program_id(0)

    @pl.when(b == 0)
    def _():
        s_s[...] = jnp.zeros((1, G), jnp.float32)
        p_s[...] = jnp.zeros((G, DIM), jnp.float32)

    oh = batch_ref[...] == lax.broadcasted_iota(jnp.int32, (BLK, G), 1)
    e = jnp.where(oh, jnp.exp(gate_ref[...] - m_ref[...]), 0.0)
    s_s[...] += jnp.sum(e, axis=0, keepdims=True)
    p_s[...] += lax.dot_general(e, h2_ref[...], (((0,), (0,)), ((), ())),
                                preferred_element_type=jnp.float32)

    @pl.when(b == NB - 1)
    def _():
        sv = s_s[...]
        rs = 1.0 / jnp.where(sv > 0, sv, 1.0)
        ii = lax.broadcasted_iota(jnp.int32, (G, G), 0)
        jj = lax.broadcasted_iota(jnp.int32, (G, G), 1)
        dm = jnp.where(ii == jj, rs, 0.0)
        pooled = jnp.dot(dm, p_s[...], preferred_element_type=jnp.float32)
        q = jnp.maximum(
            jnp.dot(pooled, wc1_ref[...], preferred_element_type=jnp.float32)
            + bc1_ref[...], 0.0)
        out_ref[...] = (jnp.dot(q, wc2_ref[...],
                                preferred_element_type=jnp.float32)
                        + bc2_ref[...])


def _kd(h2, gate, batch2, m, Wc1, bc1r, Wc2, bc2r):
    return pl.pallas_call(
        _kd_body,
        grid=(NB,),
        in_specs=[
            pl.BlockSpec((BLK, DIM), lambda b: (b, 0)),
            pl.BlockSpec((BLK, 1), lambda b: (b, 0)),
            pl.BlockSpec((BLK, 1), lambda b: (b, 0)),
            pl.BlockSpec((1, G), lambda b: (0, 0)),
            pl.BlockSpec((DIM, 128), lambda b: (0, 0)),
            pl.BlockSpec((1, 128), lambda b: (0, 0)),
            pl.BlockSpec((128, 2), lambda b: (0, 0)),
            pl.BlockSpec((1, 2), lambda b: (0, 0)),
        ],
        out_specs=pl.BlockSpec((G, 2), lambda b: (0, 0)),
        out_shape=jax.ShapeDtypeStruct((G, 2), jnp.float32),
        scratch_shapes=[
            pltpu.VMEM((1, G), jnp.float32),
            pltpu.VMEM((G, DIM), jnp.float32),
        ],
    )(h2, gate, batch2, m, Wc1, bc1r, Wc2, bc2r)


def kernel(x, edge_index, batch, emb, W1, b1, ln1_g, ln1_b, W2, b2,
           ln2_g, ln2_b, Wg1, bg1, Wg2, bg2, Wc1, bc1, Wc2, bc2):
    edges = edge_index.reshape(2 * E)
    batch2 = batch.astype(jnp.int32).reshape(N, 1)
    w1a = W1[:112]
    w1b = W1[112:]
    b1r = b1.reshape(1, DIM)
    g1r = ln1_g.reshape(1, DIM)
    be1r = ln1_b.reshape(1, DIM)
    b2r = b2.reshape(1, DIM)
    g2r = ln2_g.reshape(1, DIM)
    be2r = ln2_b.reshape(1, DIM)
    bg1r = bg1.reshape(1, 128)
    bg2r = bg2.reshape(1, 1)
    bc1r = bc1.reshape(1, 128)
    bc2r = bc2.reshape(1, 2)

    degp = _deg(jnp.ones((128, HDIM), jnp.float32), edges)
    plo1, phi1, dis = _ka(x, degp, emb, w1a, w1b)
    accp1 = _spmm(jnp.concatenate([plo1, phi1], axis=0), edges)
    plo2, phi2 = _kb(accp1, plo1, phi1, dis, b1r, g1r, be1r, W2)
    accp2 = _spmm(jnp.concatenate([plo2, phi2], axis=0), edges)
    h2, gate, m = _kc(accp2, plo2, phi2, dis, batch2, b2r, g2r, be2r,
                      Wg1, bg1r, Wg2, bg2r)
    return _kd(h2, gate, batch2, m, Wc1, bc1r, Wc2, bc2r)


# speedup vs baseline: 12.3175x; 1.0213x over previous
import jax
import jax.numpy as jnp
from jax import lax
from jax.experimental import pallas as pl
from jax.experimental.pallas import tpu as pltpu
from jax.experimental.pallas import tpu_sc as plsc

N = 10000
E = 320000
G = 64
DIM = 256
HDIM = 128
BLK = 200
NB = N // BLK

_mesh_cache = {}


def _mesh():
    if "m" not in _mesh_cache:
        _mesh_cache["m"] = plsc.VectorSubcoreMesh(
            core_axis_name="c", subcore_axis_name="s",
            num_cores=2, num_subcores=16)
    return _mesh_cache["m"]


def _deg_body(ones_hbm, edge_hbm, out_hbm, idx_d, idx_d2, rows, hist,
              dsa, dsb):
    c = lax.axis_index("c")
    s = lax.axis_index("s")

    def zero_rows(i, _):
        rows[i // 8, pl.ds((i % 8) * 16, 16)] = jnp.zeros((16,), jnp.float32)
        return 0
    lax.fori_loop(0, 1024, zero_rows, 0)

    def zero_acc(j, _):
        pltpu.sync_copy(rows.at[pl.ds(0, 104)],
                        hist.at[pl.ds(s * 624 + j * 104, 104)])
        return 0
    lax.fori_loop(0, 6, zero_acc, 0)

    @pl.when(s == 0)
    def _():
        pltpu.sync_copy(rows.at[pl.ds(0, 16)], hist.at[pl.ds(9984, 16)])

    plsc.subcore_barrier()

    pltpu.sync_copy(ones_hbm.at[pl.ds(0, 128)], rows)

    k = c * 16 + s

    def pair(j, _):
        offa = ((2 * j) * 32 + k) * 128
        offb = ((2 * j + 1) * 32 + k) * 128
        pltpu.sync_copy(edge_hbm.at[pl.ds(E + offa, 128)], idx_d)
        fa = pltpu.async_copy(rows, hist.at[idx_d], dsa, add=True)
        pltpu.sync_copy(edge_hbm.at[pl.ds(E + offb, 128)], idx_d2)
        fb = pltpu.async_copy(rows, hist.at[idx_d2], dsb, add=True)
        fa.wait()
        fb.wait()
        return 0
    lax.fori_loop(0, 39, pair, 0)

    @pl.when(k < 4)
    def _():
        off = (2496 + k) * 128
        pltpu.sync_copy(edge_hbm.at[pl.ds(E + off, 128)], idx_d)
        pltpu.sync_copy(rows, hist.at[idx_d], add=True)

    plsc.subcore_barrier()

    def copy_out(j, _):
        pltpu.sync_copy(hist.at[pl.ds(s * 624 + j * 104, 104)],
                        rows.at[pl.ds(0, 104)])
        pltpu.sync_copy(rows.at[pl.ds(0, 104)],
                        out_hbm.at[pl.ds(c * N + s * 624 + j * 104, 104)])
        return 0
    lax.fori_loop(0, 6, copy_out, 0)

    @pl.when(s == 0)
    def _():
        pltpu.sync_copy(hist.at[pl.ds(9984, 16)], rows.at[pl.ds(0, 16)])
        pltpu.sync_copy(rows.at[pl.ds(0, 16)],
                        out_hbm.at[pl.ds(c * N + 9984, 16)])


def _deg(ones, edges):
    return pl.kernel(
        _deg_body,
        out_type=jax.ShapeDtypeStruct((2 * N, HDIM), jnp.float32),
        mesh=_mesh(),
        scratch_types=[
            pltpu.VMEM((128,), jnp.int32),
            pltpu.VMEM((128,), jnp.int32),
            pltpu.VMEM((128, HDIM), jnp.float32),
            pltpu.VMEM_SHARED((N, HDIM), jnp.float32),
            pltpu.SemaphoreType.DMA,
            pltpu.SemaphoreType.DMA,
        ],
    )(ones, edges)


def _spmm_body(tbl_hbm, edge_hbm, out_hbm,
               idx_sa, idx_da, idx_sb, idx_db, rows_a, rows_b,
               idx_src_t, idx_dst_t, rows_t, acc, gsa, gsb, ssa, ssb):
    c = lax.axis_index("c")
    s = lax.axis_index("s")

    def zero_rows(i, _):
        rows_a[i // 8, pl.ds((i % 8) * 16, 16)] = jnp.zeros((16,), jnp.float32)
        return 0
    lax.fori_loop(0, 1024, zero_rows, 0)

    def zero_acc(j, _):
        pltpu.sync_copy(rows_a.at[pl.ds(0, 104)],
                        acc.at[pl.ds(s * 624 + j * 104, 104)])
        return 0
    lax.fori_loop(0, 6, zero_acc, 0)

    @pl.when(s == 0)
    def _():
        pltpu.sync_copy(rows_a.at[pl.ds(0, 16)], acc.at[pl.ds(9984, 16)])

    plsc.subcore_barrier()

    base = s * 20000
    cn = c * N

    def bias(idx_ref):
        def one(k, _):
            idx_ref[pl.ds(k * 16, 16)] = idx_ref[pl.ds(k * 16, 16)] + cn
            return 0
        lax.fori_loop(0, 8, one, 0)

    def pair(j, _):
        offa = base + (2 * j) * 128
        offb = offa + 128
        pltpu.sync_copy(edge_hbm.at[pl.ds(offa, 128)], idx_sa)
        pltpu.sync_copy(edge_hbm.at[pl.ds(E + offa, 128)], idx_da)
        bias(idx_sa)
        fa = pltpu.async_copy(tbl_hbm.at[idx_sa], rows_a, gsa)
        pltpu.sync_copy(edge_hbm.at[pl.ds(offb, 128)], idx_sb)
        pltpu.sync_copy(edge_hbm.at[pl.ds(E + offb, 128)], idx_db)
        bias(idx_sb)
        fb = pltpu.async_copy(tbl_hbm.at[idx_sb], rows_b, gsb)
        fa.wait()
        sa = pltpu.async_copy(rows_a, acc.at[idx_da], ssa, add=True)
        fb.wait()
        sb = pltpu.async_copy(rows_b, acc.at[idx_db], ssb, add=True)
        sa.wait()
        sb.wait()
        return 0
    lax.fori_loop(0, 78, pair, 0)

    off = base + 19968
    pltpu.sync_copy(edge_hbm.at[pl.ds(off, 32)], idx_src_t)
    pltpu.sync_copy(edge_hbm.at[pl.ds(E + off, 32)], idx_dst_t)

    def bias_t(k, _):
        idx_src_t[pl.ds(k * 16, 16)] = idx_src_t[pl.ds(k * 16, 16)] + cn
        return 0
    lax.fori_loop(0, 2, bias_t, 0)
    pltpu.sync_copy(tbl_hbm.at[idx_src_t], rows_t)
    pltpu.sync_copy(rows_t, acc.at[idx_dst_t], add=True)
    plsc.subcore_barrier()

    def copy_out(j, _):
        pltpu.sync_copy(acc.at[pl.ds(s * 624 + j * 104, 104)],
                        rows_a.at[pl.ds(0, 104)])
        pltpu.sync_copy(rows_a.at[pl.ds(0, 104)],
                        out_hbm.at[pl.ds(c * N + s * 624 + j * 104, 104)])
        return 0
    lax.fori_loop(0, 6, copy_out, 0)

    @pl.when(s == 0)
    def _():
        pltpu.sync_copy(acc.at[pl.ds(9984, 16)], rows_t.at[pl.ds(0, 16)])
        pltpu.sync_copy(rows_t.at[pl.ds(0, 16)],
                        out_hbm.at[pl.ds(c * N + 9984, 16)])


def _spmm(tbl, edges):
    return pl.kernel(
        _spmm_body,
        out_type=jax.ShapeDtypeStruct((2 * N, HDIM), jnp.float32),
        mesh=_mesh(),
        scratch_types=[
            pltpu.VMEM((128,), jnp.int32),
            pltpu.VMEM((128,), jnp.int32),
            pltpu.VMEM((128,), jnp.int32),
            pltpu.VMEM((128,), jnp.int32),
            pltpu.VMEM((128, HDIM), jnp.float32),
            pltpu.VMEM((128, HDIM), jnp.float32),
            pltpu.VMEM((32,), jnp.int32),
            pltpu.VMEM((32,), jnp.int32),
            pltpu.VMEM((32, HDIM), jnp.float32),
            pltpu.VMEM_SHARED((N, HDIM), jnp.float32),
            pltpu.SemaphoreType.DMA,
            pltpu.SemaphoreType.DMA,
            pltpu.SemaphoreType.DMA,
            pltpu.SemaphoreType.DMA,
        ],
    )(tbl, edges)


def _ka_body(x_ref, d0_ref, d1_ref, emb_ref, w1a_ref, w1b_ref,
             lo_ref, hi_ref, dis_ref):
    nt = x_ref[:, 0:1].astype(jnp.int32)
    feats = x_ref[:, 1:]
    oh = (nt == lax.broadcasted_iota(jnp.int32, (BLK, 400), 1)
          ).astype(jnp.float32)
    embp = jnp.dot(oh, emb_ref[...], preferred_element_type=jnp.float32)
    hw = (jnp.dot(feats, w1a_ref[...], preferred_element_type=jnp.float32)
          + jnp.dot(embp, w1b_ref[...], preferred_element_type=jnp.float32))
    deg = 1.0 + d0_ref[:, 0:1] + d1_ref[:, 0:1]
    dis = lax.rsqrt(deg)
    hwp = dis * hw
    lo_ref[...] = hwp[:, :HDIM]
    hi_ref[...] = hwp[:, HDIM:]
    dis_ref[...] = dis


def _ka(x, degp, emb, w1a, w1b):
    return pl.pallas_call(
        _ka_body,
        grid=(NB,),
        in_specs=[
            pl.BlockSpec((BLK, 113), lambda b: (b, 0)),
            pl.BlockSpec((BLK, HDIM), lambda b: (b, 0)),
            pl.BlockSpec((BLK, HDIM), lambda b: (b + NB, 0)),
            pl.BlockSpec((400, 16), lambda b: (0, 0)),
            pl.BlockSpec((112, DIM), lambda b: (0, 0)),
            pl.BlockSpec((16, DIM), lambda b: (0, 0)),
        ],
        out_specs=[
            pl.BlockSpec((BLK, HDIM), lambda b: (b, 0)),
            pl.BlockSpec((BLK, HDIM), lambda b: (b, 0)),
            pl.BlockSpec((BLK, 1), lambda b: (b, 0)),
        ],
        out_shape=[
            jax.ShapeDtypeStruct((N, HDIM), jnp.float32),
            jax.ShapeDtypeStruct((N, HDIM), jnp.float32),
            jax.ShapeDtypeStruct((N, 1), jnp.float32),
        ],
    )(x, degp, degp, emb, w1a, w1b)


def _kb_body(alo_ref, ahi_ref, plo_ref, phi_ref, dis_ref, b1_ref, g_ref,
             be_ref, w2_ref, lo_ref, hi_ref):
    dis = dis_ref[...]
    acc = jnp.concatenate([alo_ref[...], ahi_ref[...]], axis=1)
    hwp = jnp.concatenate([plo_ref[...], phi_ref[...]], axis=1)
    t = dis * (acc + hwp) + b1_ref[...]
    mu = jnp.mean(t, axis=1, keepdims=True)
    var = jnp.mean((t - mu) ** 2, axis=1, keepdims=True)
    y = (t - mu) * lax.rsqrt(var + 1e-5) * g_ref[...] + be_ref[...]
    h = jnp.maximum(y, 0.0)
    hwp2 = dis * jnp.dot(h, w2_ref[...], preferred_element_type=jnp.float32)
    lo_ref[...] = hwp2[:, :HDIM]
    hi_ref[...] = hwp2[:, HDIM:]


def _kb(accp, plo, phi, dis, b1r, g1r, be1r, W2):
    return pl.pallas_call(
        _kb_body,
        grid=(NB,),
        in_specs=[
            pl.BlockSpec((BLK, HDIM), lambda b: (b, 0)),
            pl.BlockSpec((BLK, HDIM), lambda b: (b + NB, 0)),
            pl.BlockSpec((BLK, HDIM), lambda b: (b, 0)),
            pl.BlockSpec((BLK, HDIM), lambda b: (b, 0)),
            pl.BlockSpec((BLK, 1), lambda b: (b, 0)),
            pl.BlockSpec((1, DIM), lambda b: (0, 0)),
            pl.BlockSpec((1, DIM), lambda b: (0, 0)),
            pl.BlockSpec((1, DIM), lambda b: (0, 0)),
            pl.BlockSpec((DIM, DIM), lambda b: (0, 0)),
        ],
        out_specs=[
            pl.BlockSpec((BLK, HDIM), lambda b: (b, 0)),
            pl.BlockSpec((BLK, HDIM), lambda b: (b, 0)),
        ],
        out_shape=[
            jax.ShapeDtypeStruct((N, HDIM), jnp.float32),
            jax.ShapeDtypeStruct((N, HDIM), jnp.float32),
        ],
    )(accp, accp, plo, phi, dis, b1r, g1r, be1r, W2)


def _kc_body(alo_ref, ahi_ref, plo_ref, phi_ref, dis_ref, batch_ref, b2_ref,
             g_ref, be_ref, wg1_ref, bg1_ref, wg2_ref, bg2_ref,
             h2_ref, gate_ref, m_ref):
    b = pl.program_id(0)
    dis = dis_ref[...]
    acc = jnp.concatenate([alo_ref[...], ahi_ref[...]], axis=1)
    hwp = jnp.concatenate([plo_ref[...], phi_ref[...]], axis=1)
    t = dis * (acc + hwp) + b2_ref[...]
    mu = jnp.mean(t, axis=1, keepdims=True)
    var = jnp.mean((t - mu) ** 2, axis=1, keepdims=True)
    y = (t - mu) * lax.rsqrt(var + 1e-5) * g_ref[...] + be_ref[...]
    h2 = jnp.maximum(y, 0.0)
    h2_ref[...] = h2
    g1 = jnp.maximum(
        jnp.dot(h2, wg1_ref[...], preferred_element_type=jnp.float32)
        + bg1_ref[...], 0.0)
    gate = (jnp.dot(g1, wg2_ref[...], preferred_element_type=jnp.float32)
            + bg2_ref[...])
    gate_ref[...] = gate
    oh = batch_ref[...] == lax.broadcasted_iota(jnp.int32, (BLK, G), 1)
    masked = jnp.where(oh, gate, -jnp.inf)
    bm = jnp.max(masked, axis=0, keepdims=True)

    @pl.when(b == 0)
    def _():
        m_ref[...] = jnp.full((1, G), -jnp.inf, jnp.float32)

    m_ref[...] = jnp.maximum(m_ref[...], bm)


def _kc(accp, plo, phi, dis, batch2, b2r, g2r, be2r, Wg1, bg1r, Wg2, bg2r):
    return pl.pallas_call(
        _kc_body,
        grid=(NB,),
        in_specs=[
            pl.BlockSpec((BLK, HDIM), lambda b: (b, 0)),
            pl.BlockSpec((BLK, HDIM), lambda b: (b + NB, 0)),
            pl.BlockSpec((BLK, HDIM), lambda b: (b, 0)),
            pl.BlockSpec((BLK, HDIM), lambda b: (b, 0)),
            pl.BlockSpec((BLK, 1), lambda b: (b, 0)),
            pl.BlockSpec((BLK, 1), lambda b: (b, 0)),
            pl.BlockSpec((1, DIM), lambda b: (0, 0)),
            pl.BlockSpec((1, DIM), lambda b: (0, 0)),
            pl.BlockSpec((1, DIM), lambda b: (0, 0)),
            pl.BlockSpec((DIM, 128), lambda b: (0, 0)),
            pl.BlockSpec((1, 128), lambda b: (0, 0)),
            pl.BlockSpec((128, 1), lambda b: (0, 0)),
            pl.BlockSpec((1, 1), lambda b: (0, 0)),
        ],
        out_specs=[
            pl.BlockSpec((BLK, DIM), lambda b: (b, 0)),
            pl.BlockSpec((BLK, 1), lambda b: (b, 0)),
            pl.BlockSpec((1, G), lambda b: (0, 0)),
        ],
        out_shape=[
            jax.ShapeDtypeStruct((N, DIM), jnp.float32),
            jax.ShapeDtypeStruct((N, 1), jnp.float32),
            jax.ShapeDtypeStruct((1, G), jnp.float32),
        ],
    )(accp, accp, plo, phi, dis, batch2, b2r, g2r, be2r,
      Wg1, bg1r, Wg2, bg2r)


def _kd_body(h2_ref, gate_ref, batch_ref, m_ref, wc1_ref, bc1_ref,
             wc2_ref, bc2_ref, out_ref, s_s, p_s):
    b = pl.program_id(0)

    @pl.when(b == 0)
    def _():
        s_s[...] = jnp.zeros((1, G), jnp.float32)
        p_s[...] = jnp.zeros((G, DIM), jnp.float32)

    oh = batch_ref[...] == lax.broadcasted_iota(jnp.int32, (BLK, G), 1)
    e = jnp.where(oh, jnp.exp(gate_ref[...] - m_ref[...]), 0.0)
    s_s[...] += jnp.sum(e, axis=0, keepdims=True)
    p_s[...] += lax.dot_general(e, h2_ref[...], (((0,), (0,)), ((), ())),
                                preferred_element_type=jnp.float32)

    @pl.when(b == NB - 1)
    def _():
        sv = s_s[...]
        rs = 1.0 / jnp.where(sv > 0, sv, 1.0)
        ii = lax.broadcasted_iota(jnp.int32, (G, G), 0)
        jj = lax.broadcasted_iota(jnp.int32, (G, G), 1)
        dm = jnp.where(ii == jj, rs, 0.0)
        pooled = jnp.dot(dm, p_s[...], preferred_element_type=jnp.float32)
        q = jnp.maximum(
            jnp.dot(pooled, wc1_ref[...], preferred_element_type=jnp.float32)
            + bc1_ref[...], 0.0)
        out_ref[...] = (jnp.dot(q, wc2_ref[...],
                                preferred_element_type=jnp.float32)
                        + bc2_ref[...])


def _kd(h2, gate, batch2, m, Wc1, bc1r, Wc2, bc2r):
    return pl.pallas_call(
        _kd_body,
        grid=(NB,),
        in_specs=[
            pl.BlockSpec((BLK, DIM), lambda b: (b, 0)),
            pl.BlockSpec((BLK, 1), lambda b: (b, 0)),
            pl.BlockSpec((BLK, 1), lambda b: (b, 0)),
            pl.BlockSpec((1, G), lambda b: (0, 0)),
            pl.BlockSpec((DIM, 128), lambda b: (0, 0)),
            pl.BlockSpec((1, 128), lambda b: (0, 0)),
            pl.BlockSpec((128, 2), lambda b: (0, 0)),
            pl.BlockSpec((1, 2), lambda b: (0, 0)),
        ],
        out_specs=pl.BlockSpec((G, 2), lambda b: (0, 0)),
        out_shape=jax.ShapeDtypeStruct((G, 2), jnp.float32),
        scratch_shapes=[
            pltpu.VMEM((1, G), jnp.float32),
            pltpu.VMEM((G, DIM), jnp.float32),
        ],
    )(h2, gate, batch2, m, Wc1, bc1r, Wc2, bc2r)


def kernel(x, edge_index, batch, emb, W1, b1, ln1_g, ln1_b, W2, b2,
           ln2_g, ln2_b, Wg1, bg1, Wg2, bg2, Wc1, bc1, Wc2, bc2):
    edges = edge_index.reshape(2 * E)
    batch2 = batch.astype(jnp.int32).reshape(N, 1)
    w1a = W1[:112]
    w1b = W1[112:]
    b1r = b1.reshape(1, DIM)
    g1r = ln1_g.reshape(1, DIM)
    be1r = ln1_b.reshape(1, DIM)
    b2r = b2.reshape(1, DIM)
    g2r = ln2_g.reshape(1, DIM)
    be2r = ln2_b.reshape(1, DIM)
    bg1r = bg1.reshape(1, 128)
    bg2r = bg2.reshape(1, 1)
    bc1r = bc1.reshape(1, 128)
    bc2r = bc2.reshape(1, 2)

    degp = _deg(jnp.ones((128, HDIM), jnp.float32), edges)
    plo1, phi1, dis = _ka(x, degp, emb, w1a, w1b)
    accp1 = _spmm(jnp.concatenate([plo1, phi1], axis=0), edges)
    plo2, phi2 = _kb(accp1, plo1, phi1, dis, b1r, g1r, be1r, W2)
    accp2 = _spmm(jnp.concatenate([plo2, phi2], axis=0), edges)
    h2, gate, m = _kc(accp2, plo2, phi2, dis, batch2, b2r, g2r, be2r,
                      Wg1, bg1r, Wg2, bg2r)
    return _kd(h2, gate, batch2, m, Wc1, bc1r, Wc2, bc2r)
